# baseline (speedup 1.0000x reference)
import jax
import jax.numpy as jnp
from jax.experimental import pallas as pl

_N = 10000
_D = 512
_BM = 400


def _gcn_matmul_kernel(adj_ref, emb_ref, out_ref):
    a = adj_ref[...].astype(jnp.bfloat16)
    out_ref[...] = jnp.dot(a, emb_ref[...], preferred_element_type=jnp.float32)


def kernel(adj, embeds):
    emb_bf16 = embeds.astype(jnp.bfloat16)
    return pl.pallas_call(
        _gcn_matmul_kernel,
        grid=(_N // _BM,),
        in_specs=[
            pl.BlockSpec((_BM, _N), lambda i: (i, 0)),
            pl.BlockSpec((_N, _D), lambda i: (0, 0)),
        ],
        out_specs=pl.BlockSpec((_BM, _D), lambda i: (i, 0)),
        out_shape=jax.ShapeDtypeStruct((_N, _D), jnp.float32),
    )(adj, emb_bf16)
